# baseline (speedup 1.0000x reference)
import jax
import jax.numpy as jnp
from jax import lax
from jax.experimental import pallas as pl
from jax.experimental.pallas import tpu as pltpu
from jax.experimental.pallas import tpu_sc as plsc

B = 16384
EMB = 64
NC = 2
NS = 16
L = 16
NW = NC * NS
BPW = B // NW
NCHUNK = 4
CROWS = BPW // NCHUNK
NGROUPS = BPW // L

GOFF = 8
YOFF = 40
PROJ_N = 144


def _tc_proj_body(ut, gt, yt, w0, w2, w3, b, proj):
    proj[0:8, :] = jnp.dot(ut[...], w0[...], preferred_element_type=jnp.float32)
    proj[8:40, :] = jnp.dot(gt[...], w2[...], preferred_element_type=jnp.float32)
    proj[40:144, :] = jnp.dot(yt[...], w3[...],
                              preferred_element_type=jnp.float32) + b[0, 0]


def _sc_body(midx_hbm, uidx_hbm, gidx_hbm, yidx_hbm, proj_hbm, wm_hbm, mtab_hbm,
             out_hbm, midx_v, uidx_v, gidx_v, yidx_v, proj_v, wm_v, rows_v,
             out_v, sem):
    wid = lax.axis_index("s") * NC + lax.axis_index("c")
    base = wid * BPW

    pltpu.sync_copy(midx_hbm.at[pl.ds(wid * NCHUNK, NCHUNK)], midx_v)
    pltpu.sync_copy(uidx_hbm.at[pl.ds(base, BPW)], uidx_v)
    pltpu.sync_copy(gidx_hbm.at[pl.ds(base, BPW)], gidx_v)
    pltpu.sync_copy(yidx_hbm.at[pl.ds(base, BPW)], yidx_v)
    pltpu.sync_copy(proj_hbm, proj_v)
    pltpu.sync_copy(wm_hbm, wm_v)

    descs = [
        pltpu.async_copy(mtab_hbm.at[midx_v.at[j]],
                         rows_v.at[pl.ds(j * CROWS, CROWS)], sem)
        for j in range(NCHUNK)
    ]
    for d in descs:
        d.wait()

    riota = lax.iota(jnp.int32, L)

    def group(g, carry):
        row0 = g * L
        rvec = riota + row0
        acc = jnp.zeros((L,), jnp.float32)
        for cb in range(EMB // L):
            wv = wm_v[pl.ds(cb * L, L)]
            for j in range(L):
                c = cb * L + j
                col = plsc.load_gather(
                    rows_v, [rvec, jnp.full((L,), c, jnp.int32)])
                acc = acc + col * wv[j]
        uvec = uidx_v[pl.ds(row0, L)]
        gvec = gidx_v[pl.ds(row0, L)] + GOFF
        yvec = yidx_v[pl.ds(row0, L)] + YOFF
        acc = acc + plsc.load_gather(proj_v, [uvec])
        acc = acc + plsc.load_gather(proj_v, [gvec])
        acc = acc + plsc.load_gather(proj_v, [yvec])
        out_v[pl.ds(row0, L)] = acc
        return carry

    lax.fori_loop(0, NGROUPS, group, 0)
    pltpu.sync_copy(out_v, out_hbm.at[pl.ds(base, BPW)])


def kernel(user, movie, genre, year, user_table, movie_table, genre_table,
           year_table, fc_w, fc_b):
    user = user.astype(jnp.int32)
    movie = movie.astype(jnp.int32)
    genre = genre.astype(jnp.int32)
    year = year.astype(jnp.int32)

    w0 = fc_w[0:64]
    wm = fc_w[64:128, 0]
    w2 = fc_w[128:192]
    w3 = fc_w[192:256]
    ut = jnp.pad(user_table, ((0, 3), (0, 0)))
    gt = jnp.pad(genre_table, ((0, 1), (0, 0)))
    yt = jnp.pad(year_table, ((0, 3), (0, 0)))

    proj = pl.pallas_call(
        _tc_proj_body,
        out_shape=jax.ShapeDtypeStruct((PROJ_N, 1), jnp.float32),
    )(ut, gt, yt, w0, w2, w3, fc_b.reshape(1, 1))
    proj1 = proj[:, 0]

    sc = pl.kernel(
        _sc_body,
        mesh=plsc.VectorSubcoreMesh(core_axis_name="c", subcore_axis_name="s"),
        compiler_params=pltpu.CompilerParams(needs_layout_passes=False,
                                             use_tc_tiling_on_sc=False),
        out_type=jax.ShapeDtypeStruct((B,), jnp.float32),
        scratch_types=[
            pltpu.VMEM((NCHUNK, CROWS), jnp.int32),
            pltpu.VMEM((BPW,), jnp.int32),
            pltpu.VMEM((BPW,), jnp.int32),
            pltpu.VMEM((BPW,), jnp.int32),
            pltpu.VMEM((PROJ_N,), jnp.float32),
            pltpu.VMEM((EMB,), jnp.float32),
            pltpu.VMEM((BPW, EMB), jnp.float32),
            pltpu.VMEM((BPW,), jnp.float32),
            pltpu.SemaphoreType.DMA,
        ],
    )
    out = sc(movie.reshape(NW * NCHUNK, CROWS), user, genre, year, proj1, wm,
             movie_table)
    return out.reshape(B, 1)


# speedup vs baseline: 4.0229x; 4.0229x over previous
import jax
import jax.numpy as jnp
from jax import lax
from jax.experimental import pallas as pl
from jax.experimental.pallas import tpu as pltpu
from jax.experimental.pallas import tpu_sc as plsc

B = 16384
EMB = 64
NUM_MOVIES = 1000000
NC = 2
NS = 16
L = 16
NW = NC * NS
BPW = B // NW
NCHUNK = 4
CROWS = BPW // NCHUNK
NGROUPS = BPW // L

MV_BLK = 8192
MV_GRID = -(-NUM_MOVIES // MV_BLK)

GOFF = 8
YOFF = 40
PROJ_N = 144


def _tc_proj_body(ut, gt, yt, w0, w2, w3, b, proj):
    proj[0:8, :] = jnp.dot(ut[...], w0[...], preferred_element_type=jnp.float32)
    proj[8:40, :] = jnp.dot(gt[...], w2[...], preferred_element_type=jnp.float32)
    proj[40:144, :] = jnp.dot(yt[...], w3[...],
                              preferred_element_type=jnp.float32) + b[0, 0]


def _tc_mv_body(wrow, xt, out):
    out[...] = jnp.dot(wrow[...], xt[...], preferred_element_type=jnp.float32)


def _sc_body(hi_hbm, lo_hbm, uidx_hbm, gidx_hbm, yidx_hbm, proj_hbm, p_hbm,
             out_hbm, hi_v, lo_v, uidx_v, gidx_v, yidx_v, proj_v, prows_v,
             out_v, sem):
    wid = lax.axis_index("s") * NC + lax.axis_index("c")
    base = wid * BPW

    pltpu.sync_copy(hi_hbm.at[pl.ds(wid * NCHUNK, NCHUNK)], hi_v)
    pltpu.sync_copy(lo_hbm.at[pl.ds(base, BPW)], lo_v)
    pltpu.sync_copy(uidx_hbm.at[pl.ds(base, BPW)], uidx_v)
    pltpu.sync_copy(gidx_hbm.at[pl.ds(base, BPW)], gidx_v)
    pltpu.sync_copy(yidx_hbm.at[pl.ds(base, BPW)], yidx_v)
    pltpu.sync_copy(proj_hbm, proj_v)

    descs = [
        pltpu.async_copy(p_hbm.at[hi_v.at[j]],
                         prows_v.at[pl.ds(j * CROWS, CROWS)], sem)
        for j in range(NCHUNK)
    ]
    for d in descs:
        d.wait()

    riota = lax.iota(jnp.int32, L)

    def group(g, carry):
        row0 = g * L
        rvec = riota + row0
        lovec = lo_v[pl.ds(row0, L)]
        acc = plsc.load_gather(prows_v, [rvec, lovec])
        uvec = uidx_v[pl.ds(row0, L)]
        gvec = gidx_v[pl.ds(row0, L)] + GOFF
        yvec = yidx_v[pl.ds(row0, L)] + YOFF
        acc = acc + plsc.load_gather(proj_v, [uvec])
        acc = acc + plsc.load_gather(proj_v, [gvec])
        acc = acc + plsc.load_gather(proj_v, [yvec])
        out_v[pl.ds(row0, L)] = acc
        return carry

    lax.fori_loop(0, NGROUPS, group, 0)
    pltpu.sync_copy(out_v, out_hbm.at[pl.ds(base, BPW)])


def kernel(user, movie, genre, year, user_table, movie_table, genre_table,
           year_table, fc_w, fc_b):
    user = user.astype(jnp.int32)
    movie = movie.astype(jnp.int32)
    genre = genre.astype(jnp.int32)
    year = year.astype(jnp.int32)

    w0 = fc_w[0:64]
    wm = fc_w[64:128]
    w2 = fc_w[128:192]
    w3 = fc_w[192:256]
    ut = jnp.pad(user_table, ((0, 3), (0, 0)))
    gt = jnp.pad(genre_table, ((0, 1), (0, 0)))
    yt = jnp.pad(year_table, ((0, 3), (0, 0)))

    proj = pl.pallas_call(
        _tc_proj_body,
        out_shape=jax.ShapeDtypeStruct((PROJ_N, 1), jnp.float32),
    )(ut, gt, yt, w0, w2, w3, fc_b.reshape(1, 1))
    proj1 = proj[:, 0]

    p = pl.pallas_call(
        _tc_mv_body,
        grid=(MV_GRID,),
        in_specs=[
            pl.BlockSpec((1, EMB), lambda i: (0, 0)),
            pl.BlockSpec((EMB, MV_BLK), lambda i: (0, i)),
        ],
        out_specs=pl.BlockSpec((1, MV_BLK), lambda i: (0, i)),
        out_shape=jax.ShapeDtypeStruct((1, MV_GRID * MV_BLK), jnp.float32),
    )(wm.reshape(1, EMB), movie_table.T)
    p16 = p.reshape(MV_GRID * MV_BLK // 16, 16)

    hi = lax.shift_right_logical(movie, 4).reshape(NW * NCHUNK, CROWS)
    lo = movie & 15

    sc = pl.kernel(
        _sc_body,
        mesh=plsc.VectorSubcoreMesh(core_axis_name="c", subcore_axis_name="s"),
        compiler_params=pltpu.CompilerParams(needs_layout_passes=False,
                                             use_tc_tiling_on_sc=False),
        out_type=jax.ShapeDtypeStruct((B,), jnp.float32),
        scratch_types=[
            pltpu.VMEM((NCHUNK, CROWS), jnp.int32),
            pltpu.VMEM((BPW,), jnp.int32),
            pltpu.VMEM((BPW,), jnp.int32),
            pltpu.VMEM((BPW,), jnp.int32),
            pltpu.VMEM((BPW,), jnp.int32),
            pltpu.VMEM((PROJ_N,), jnp.float32),
            pltpu.VMEM((BPW, L), jnp.float32),
            pltpu.VMEM((BPW,), jnp.float32),
            pltpu.SemaphoreType.DMA,
        ],
    )
    out = sc(hi, lo, user, genre, year, proj1, p16)
    return out.reshape(B, 1)


# speedup vs baseline: 5.4847x; 1.3634x over previous
import jax
import jax.numpy as jnp
from jax import lax
from jax.experimental import pallas as pl
from jax.experimental.pallas import tpu as pltpu
from jax.experimental.pallas import tpu_sc as plsc

B = 16384
EMB = 64
NUM_MOVIES = 1000000
NC = 2
NS = 16
L = 16
NW = NC * NS
BPW = B // NW
NCHUNK = 4
CROWS = BPW // NCHUNK
NGROUPS = BPW // L

MV_BLK = 16384
MV_GRID = -(-NUM_MOVIES // MV_BLK)

UOFF = 0
GOFF = 16
YOFF = 48
PROJ_N = 160

_SC_PARAMS = pltpu.CompilerParams(needs_layout_passes=False,
                                  use_tc_tiling_on_sc=False)
_SC_MESH = dict(core_axis_name="c", subcore_axis_name="s")


def _tc_mv_body(wrow, xt, out):
    out[...] = jnp.dot(wrow[...], xt[...], preferred_element_type=jnp.float32)


def _sc_proj_body(ut_hbm, gt_hbm, yt_hbm, w_hbm, b_hbm, proj_hbm,
                  tab_v, w_v, b_v, proj_v):
    wid = lax.axis_index("s") * NC + lax.axis_index("c")

    @pl.when(wid == 0)
    def _():
        pltpu.sync_copy(ut_hbm, tab_v.at[pl.ds(0, 5)])
        pltpu.sync_copy(gt_hbm, tab_v.at[pl.ds(16, 31)])
        pltpu.sync_copy(yt_hbm, tab_v.at[pl.ds(48, 101)])
        pltpu.sync_copy(w_hbm, w_v)
        pltpu.sync_copy(b_hbm, b_v)
        riota = lax.iota(jnp.int32, L)
        bvec = b_v[pl.ds(0, L)]

        specs = [(UOFF, 0, 5, False), (GOFF, 128, 31, False),
                 (YOFF, 192, 101, True)]
        for toff, woff, nrows, add_b in specs:
            for g in range(-(-nrows // L)):
                rvec = jnp.minimum(riota + g * L, nrows - 1) + toff
                acc = jnp.zeros((L,), jnp.float32)
                for cb in range(EMB // L):
                    wv = w_v[pl.ds(woff + cb * L, L)]
                    for j in range(L):
                        col = plsc.load_gather(
                            tab_v,
                            [rvec, jnp.full((L,), cb * L + j, jnp.int32)])
                        acc = acc + col * wv[j]
                if add_b:
                    acc = acc + bvec
                proj_v[pl.ds(toff + g * L, L)] = acc
        pltpu.sync_copy(proj_v, proj_hbm)


def _sc_body(mflat_hbm, uidx_hbm, gidx_hbm, yidx_hbm, proj_hbm,
             p_hbm, out_hbm, hi_v, mflat_v, uidx_v, gidx_v, yidx_v,
             proj_v, prows_v, out_v, sem):
    wid = lax.axis_index("s") * NC + lax.axis_index("c")
    base = wid * BPW

    pltpu.sync_copy(mflat_hbm.at[pl.ds(base, BPW)], mflat_v)
    pltpu.sync_copy(uidx_hbm.at[pl.ds(base, BPW)], uidx_v)
    pltpu.sync_copy(gidx_hbm.at[pl.ds(base, BPW)], gidx_v)
    pltpu.sync_copy(yidx_hbm.at[pl.ds(base, BPW)], yidx_v)
    pltpu.sync_copy(proj_hbm, proj_v)

    for j in range(NCHUNK):
        for k in range(CROWS // L):
            hi_v[j, pl.ds(k * L, L)] = lax.shift_right_logical(
                mflat_v[pl.ds(j * CROWS + k * L, L)], 4)

    descs = [
        pltpu.async_copy(p_hbm.at[hi_v.at[j]],
                         prows_v.at[pl.ds(j * CROWS, CROWS)], sem)
        for j in range(NCHUNK)
    ]
    for d in descs:
        d.wait()

    riota = lax.iota(jnp.int32, L)

    def group(g, carry):
        row0 = g * L
        rvec = riota + row0
        lovec = mflat_v[pl.ds(row0, L)] & 15
        acc = plsc.load_gather(prows_v, [rvec, lovec])
        uvec = uidx_v[pl.ds(row0, L)] + UOFF
        gvec = gidx_v[pl.ds(row0, L)] + GOFF
        yvec = yidx_v[pl.ds(row0, L)] + YOFF
        acc = acc + plsc.load_gather(proj_v, [uvec])
        acc = acc + plsc.load_gather(proj_v, [gvec])
        acc = acc + plsc.load_gather(proj_v, [yvec])
        out_v[pl.ds(row0, L)] = acc
        return carry

    lax.fori_loop(0, NGROUPS, group, 0)
    pltpu.sync_copy(out_v, out_hbm.at[pl.ds(base, BPW)])


def kernel(user, movie, genre, year, user_table, movie_table, genre_table,
           year_table, fc_w, fc_b):
    user = user.astype(jnp.int32)
    movie = movie.astype(jnp.int32)
    genre = genre.astype(jnp.int32)
    year = year.astype(jnp.int32)

    wflat = fc_w.reshape(256)
    b16 = jnp.broadcast_to(fc_b, (L,))

    proj = pl.kernel(
        _sc_proj_body,
        mesh=plsc.VectorSubcoreMesh(**_SC_MESH),
        compiler_params=_SC_PARAMS,
        out_type=jax.ShapeDtypeStruct((PROJ_N,), jnp.float32),
        scratch_types=[
            pltpu.VMEM((PROJ_N, EMB), jnp.float32),
            pltpu.VMEM((256,), jnp.float32),
            pltpu.VMEM((L,), jnp.float32),
            pltpu.VMEM((PROJ_N,), jnp.float32),
        ],
    )(user_table, genre_table, year_table, wflat, b16)

    p = pl.pallas_call(
        _tc_mv_body,
        grid=(MV_GRID,),
        in_specs=[
            pl.BlockSpec((1, EMB), lambda i: (0, 0)),
            pl.BlockSpec((EMB, MV_BLK), lambda i: (0, i)),
        ],
        out_specs=pl.BlockSpec((1, MV_BLK), lambda i: (0, i)),
        out_shape=jax.ShapeDtypeStruct((1, MV_GRID * MV_BLK), jnp.float32),
    )(wflat[64:128].reshape(1, EMB), movie_table.T)
    p16 = p.reshape(MV_GRID * MV_BLK // 16, 16)

    sc = pl.kernel(
        _sc_body,
        mesh=plsc.VectorSubcoreMesh(**_SC_MESH),
        compiler_params=_SC_PARAMS,
        out_type=jax.ShapeDtypeStruct((B,), jnp.float32),
        scratch_types=[
            pltpu.VMEM((NCHUNK, CROWS), jnp.int32),
            pltpu.VMEM((BPW,), jnp.int32),
            pltpu.VMEM((BPW,), jnp.int32),
            pltpu.VMEM((BPW,), jnp.int32),
            pltpu.VMEM((BPW,), jnp.int32),
            pltpu.VMEM((PROJ_N,), jnp.float32),
            pltpu.VMEM((BPW, L), jnp.float32),
            pltpu.VMEM((BPW,), jnp.float32),
            pltpu.SemaphoreType.DMA,
        ],
    )
    out = sc(movie, user, genre, year, proj, p16)
    return out.reshape(B, 1)


# speedup vs baseline: 5.5580x; 1.0134x over previous
import jax
import jax.numpy as jnp
from jax import lax
from jax.experimental import pallas as pl
from jax.experimental.pallas import tpu as pltpu
from jax.experimental.pallas import tpu_sc as plsc

B = 16384
EMB = 64
NUM_MOVIES = 1000000
NC = 2
NS = 16
L = 16
NW = NC * NS
BPW = B // NW
NCHUNK = 4
CROWS = BPW // NCHUNK
NGROUPS = BPW // L

MV_BLK = 16384
MV_GRID = -(-NUM_MOVIES // MV_BLK)

UOFF = 0
GOFF = 16
YOFF = 48
PROJ_N = 160

_SC_PARAMS = pltpu.CompilerParams(needs_layout_passes=False,
                                  use_tc_tiling_on_sc=False)
_SC_MESH = dict(core_axis_name="c", subcore_axis_name="s")


def _tc_mv_body(w_all, xt, utT, gtT, ytT, b, out, proj):
    out[...] = jnp.dot(w_all[0:1, 64:128], xt[...],
                       preferred_element_type=jnp.float32)

    @pl.when(pl.program_id(0) == 0)
    def _():
        proj[0:1, UOFF:UOFF + 5] = jnp.dot(
            w_all[0:1, 0:64], utT[...], preferred_element_type=jnp.float32)
        proj[0:1, GOFF:GOFF + 31] = jnp.dot(
            w_all[0:1, 128:192], gtT[...], preferred_element_type=jnp.float32)
        proj[0:1, YOFF:YOFF + 101] = jnp.dot(
            w_all[0:1, 192:256], ytT[...],
            preferred_element_type=jnp.float32) + b[0, 0]


def _sc_body(mflat_hbm, uidx_hbm, gidx_hbm, yidx_hbm, proj_hbm,
             p_hbm, out_hbm, hi_v, mflat_v, uidx_v, gidx_v, yidx_v,
             proj_v, prows_v, out_v, sem):
    wid = lax.axis_index("s") * NC + lax.axis_index("c")
    base = wid * BPW

    pltpu.sync_copy(mflat_hbm.at[pl.ds(base, BPW)], mflat_v)
    pltpu.sync_copy(uidx_hbm.at[pl.ds(base, BPW)], uidx_v)
    pltpu.sync_copy(gidx_hbm.at[pl.ds(base, BPW)], gidx_v)
    pltpu.sync_copy(yidx_hbm.at[pl.ds(base, BPW)], yidx_v)
    pltpu.sync_copy(proj_hbm, proj_v)

    for j in range(NCHUNK):
        for k in range(CROWS // L):
            hi_v[j, pl.ds(k * L, L)] = lax.shift_right_logical(
                mflat_v[pl.ds(j * CROWS + k * L, L)], 4)

    descs = [
        pltpu.async_copy(p_hbm.at[hi_v.at[j]],
                         prows_v.at[pl.ds(j * CROWS, CROWS)], sem)
        for j in range(NCHUNK)
    ]
    for d in descs:
        d.wait()

    riota = lax.iota(jnp.int32, L)

    def group(g, carry):
        row0 = g * L
        rvec = riota + row0
        lovec = mflat_v[pl.ds(row0, L)] & 15
        acc = plsc.load_gather(prows_v, [rvec, lovec])
        uvec = uidx_v[pl.ds(row0, L)] + UOFF
        gvec = gidx_v[pl.ds(row0, L)] + GOFF
        yvec = yidx_v[pl.ds(row0, L)] + YOFF
        acc = acc + plsc.load_gather(proj_v, [uvec])
        acc = acc + plsc.load_gather(proj_v, [gvec])
        acc = acc + plsc.load_gather(proj_v, [yvec])
        out_v[pl.ds(row0, L)] = acc
        return carry

    lax.fori_loop(0, NGROUPS, group, 0)
    pltpu.sync_copy(out_v, out_hbm.at[pl.ds(base, BPW)])


def kernel(user, movie, genre, year, user_table, movie_table, genre_table,
           year_table, fc_w, fc_b):
    user = user.astype(jnp.int32)
    movie = movie.astype(jnp.int32)
    genre = genre.astype(jnp.int32)
    year = year.astype(jnp.int32)

    p, proj2 = pl.pallas_call(
        _tc_mv_body,
        grid=(MV_GRID,),
        in_specs=[
            pl.BlockSpec((1, 256), lambda i: (0, 0)),
            pl.BlockSpec((EMB, MV_BLK), lambda i: (0, i)),
            pl.BlockSpec((EMB, 5), lambda i: (0, 0)),
            pl.BlockSpec((EMB, 31), lambda i: (0, 0)),
            pl.BlockSpec((EMB, 101), lambda i: (0, 0)),
            pl.BlockSpec((1, 1), lambda i: (0, 0)),
        ],
        out_specs=[
            pl.BlockSpec((1, MV_BLK), lambda i: (0, i)),
            pl.BlockSpec((1, PROJ_N), lambda i: (0, 0)),
        ],
        out_shape=[
            jax.ShapeDtypeStruct((1, MV_GRID * MV_BLK), jnp.float32),
            jax.ShapeDtypeStruct((1, PROJ_N), jnp.float32),
        ],
    )(fc_w.reshape(1, 256), movie_table.T, user_table.T, genre_table.T,
      year_table.T, fc_b.reshape(1, 1))
    p16 = p.reshape(MV_GRID * MV_BLK // 16, 16)
    proj = proj2.reshape(PROJ_N)

    sc = pl.kernel(
        _sc_body,
        mesh=plsc.VectorSubcoreMesh(**_SC_MESH),
        compiler_params=_SC_PARAMS,
        out_type=jax.ShapeDtypeStruct((B,), jnp.float32),
        scratch_types=[
            pltpu.VMEM((NCHUNK, CROWS), jnp.int32),
            pltpu.VMEM((BPW,), jnp.int32),
            pltpu.VMEM((BPW,), jnp.int32),
            pltpu.VMEM((BPW,), jnp.int32),
            pltpu.VMEM((BPW,), jnp.int32),
            pltpu.VMEM((PROJ_N,), jnp.float32),
            pltpu.VMEM((BPW, L), jnp.float32),
            pltpu.VMEM((BPW,), jnp.float32),
            pltpu.SemaphoreType.DMA,
        ],
    )
    out = sc(movie, user, genre, year, proj, p16)
    return out.reshape(B, 1)


# speedup vs baseline: 5.9352x; 1.0679x over previous
import jax
import jax.numpy as jnp
from jax import lax
from jax.experimental import pallas as pl
from jax.experimental.pallas import tpu as pltpu
from jax.experimental.pallas import tpu_sc as plsc

B = 16384
EMB = 64
NUM_MOVIES = 1000000
NC = 2
NS = 16
L = 16
NW = NC * NS
BPW = B // NW
NCHUNK = 4
CROWS = BPW // NCHUNK
NGROUPS = BPW // L

MV_BLK = 20480
MV_GRID = -(-NUM_MOVIES // MV_BLK)

UOFF = 0
GOFF = 16
YOFF = 48
PROJ_N = 160

_SC_PARAMS = pltpu.CompilerParams(needs_layout_passes=False,
                                  use_tc_tiling_on_sc=False)
_SC_MESH = dict(core_axis_name="c", subcore_axis_name="s")


def _tc_mv_body(w_all, xt, utT, gtT, ytT, b, out, proj):
    out[...] = jnp.dot(w_all[0:1, 64:128], xt[...],
                       preferred_element_type=jnp.float32)

    @pl.when(pl.program_id(0) == 0)
    def _():
        proj[0:1, UOFF:UOFF + 5] = jnp.dot(
            w_all[0:1, 0:64], utT[...], preferred_element_type=jnp.float32)
        proj[0:1, GOFF:GOFF + 31] = jnp.dot(
            w_all[0:1, 128:192], gtT[...], preferred_element_type=jnp.float32)
        proj[0:1, YOFF:YOFF + 101] = jnp.dot(
            w_all[0:1, 192:256], ytT[...],
            preferred_element_type=jnp.float32) + b[0, 0]


def _sc_body(mflat_hbm, uidx_hbm, gidx_hbm, yidx_hbm, proj_hbm,
             p_hbm, out_hbm, hi_v, mflat_v, uidx_v, gidx_v, yidx_v,
             proj_v, prows_v, out_v, sem):
    wid = lax.axis_index("s") * NC + lax.axis_index("c")
    base = wid * BPW

    pltpu.sync_copy(mflat_hbm.at[pl.ds(base, BPW)], mflat_v)
    pltpu.sync_copy(uidx_hbm.at[pl.ds(base, BPW)], uidx_v)
    pltpu.sync_copy(gidx_hbm.at[pl.ds(base, BPW)], gidx_v)
    pltpu.sync_copy(yidx_hbm.at[pl.ds(base, BPW)], yidx_v)
    pltpu.sync_copy(proj_hbm, proj_v)

    for j in range(NCHUNK):
        for k in range(CROWS // L):
            hi_v[j, pl.ds(k * L, L)] = lax.shift_right_logical(
                mflat_v[pl.ds(j * CROWS + k * L, L)], 4)

    descs = [
        pltpu.async_copy(p_hbm.at[hi_v.at[j]],
                         prows_v.at[pl.ds(j * CROWS, CROWS)], sem)
        for j in range(NCHUNK)
    ]
    for d in descs:
        d.wait()

    riota = lax.iota(jnp.int32, L)

    def group(g, carry):
        row0 = g * L
        rvec = riota + row0
        lovec = mflat_v[pl.ds(row0, L)] & 15
        acc = plsc.load_gather(prows_v, [rvec, lovec])
        uvec = uidx_v[pl.ds(row0, L)] + UOFF
        gvec = gidx_v[pl.ds(row0, L)] + GOFF
        yvec = yidx_v[pl.ds(row0, L)] + YOFF
        acc = acc + plsc.load_gather(proj_v, [uvec])
        acc = acc + plsc.load_gather(proj_v, [gvec])
        acc = acc + plsc.load_gather(proj_v, [yvec])
        out_v[pl.ds(row0, L)] = acc
        return carry

    lax.fori_loop(0, NGROUPS, group, 0)
    pltpu.sync_copy(out_v, out_hbm.at[pl.ds(base, BPW)])


def kernel(user, movie, genre, year, user_table, movie_table, genre_table,
           year_table, fc_w, fc_b):
    user = user.astype(jnp.int32)
    movie = movie.astype(jnp.int32)
    genre = genre.astype(jnp.int32)
    year = year.astype(jnp.int32)

    p, proj2 = pl.pallas_call(
        _tc_mv_body,
        grid=(MV_GRID,),
        in_specs=[
            pl.BlockSpec((1, 256), lambda i: (0, 0)),
            pl.BlockSpec((EMB, MV_BLK), lambda i: (0, i)),
            pl.BlockSpec((EMB, 5), lambda i: (0, 0)),
            pl.BlockSpec((EMB, 31), lambda i: (0, 0)),
            pl.BlockSpec((EMB, 101), lambda i: (0, 0)),
            pl.BlockSpec((1, 1), lambda i: (0, 0)),
        ],
        out_specs=[
            pl.BlockSpec((1, MV_BLK), lambda i: (0, i)),
            pl.BlockSpec((1, PROJ_N), lambda i: (0, 0)),
        ],
        out_shape=[
            jax.ShapeDtypeStruct((1, MV_GRID * MV_BLK), jnp.float32),
            jax.ShapeDtypeStruct((1, PROJ_N), jnp.float32),
        ],
    )(fc_w.reshape(1, 256), movie_table.T, user_table.T, genre_table.T,
      year_table.T, fc_b.reshape(1, 1))
    p16 = p.reshape(MV_GRID * MV_BLK // 16, 16)
    proj = proj2.reshape(PROJ_N)

    sc = pl.kernel(
        _sc_body,
        mesh=plsc.VectorSubcoreMesh(**_SC_MESH),
        compiler_params=_SC_PARAMS,
        out_type=jax.ShapeDtypeStruct((B,), jnp.float32),
        scratch_types=[
            pltpu.VMEM((NCHUNK, CROWS), jnp.int32),
            pltpu.VMEM((BPW,), jnp.int32),
            pltpu.VMEM((BPW,), jnp.int32),
            pltpu.VMEM((BPW,), jnp.int32),
            pltpu.VMEM((BPW,), jnp.int32),
            pltpu.VMEM((PROJ_N,), jnp.float32),
            pltpu.VMEM((BPW, L), jnp.float32),
            pltpu.VMEM((BPW,), jnp.float32),
            pltpu.SemaphoreType.DMA,
        ],
    )
    out = sc(movie, user, genre, year, proj, p16)
    return out.reshape(B, 1)


# speedup vs baseline: 6.1395x; 1.0344x over previous
import jax
import jax.numpy as jnp
from jax import lax
from jax.experimental import pallas as pl
from jax.experimental.pallas import tpu as pltpu
from jax.experimental.pallas import tpu_sc as plsc

B = 16384
EMB = 64
NUM_MOVIES = 1000000
NC = 2
NS = 16
L = 16
NW = NC * NS
BPW = B // NW
NCHUNK = 4
CROWS = BPW // NCHUNK
NGROUPS = BPW // L

MV_BLK = 24576
MV_GRID = -(-NUM_MOVIES // MV_BLK)

UOFF = 0
GOFF = 16
YOFF = 48
PROJ_N = 160

_SC_PARAMS = pltpu.CompilerParams(needs_layout_passes=False,
                                  use_tc_tiling_on_sc=False)
_SC_MESH = dict(core_axis_name="c", subcore_axis_name="s")


def _tc_mv_body(w_all, xt, utT, gtT, ytT, b, out, proj):
    out[...] = jnp.dot(w_all[0:1, 64:128], xt[...],
                       preferred_element_type=jnp.float32)

    @pl.when(pl.program_id(0) == 0)
    def _():
        proj[0:1, UOFF:UOFF + 5] = jnp.dot(
            w_all[0:1, 0:64], utT[...], preferred_element_type=jnp.float32)
        proj[0:1, GOFF:GOFF + 31] = jnp.dot(
            w_all[0:1, 128:192], gtT[...], preferred_element_type=jnp.float32)
        proj[0:1, YOFF:YOFF + 101] = jnp.dot(
            w_all[0:1, 192:256], ytT[...],
            preferred_element_type=jnp.float32) + b[0, 0]


def _sc_body(mflat_hbm, uidx_hbm, gidx_hbm, yidx_hbm, proj_hbm,
             p_hbm, out_hbm, hi_v, mflat_v, uidx_v, gidx_v, yidx_v,
             proj_v, prows_v, out_v, sem):
    wid = lax.axis_index("s") * NC + lax.axis_index("c")
    base = wid * BPW

    pltpu.sync_copy(mflat_hbm.at[pl.ds(base, BPW)], mflat_v)
    pltpu.sync_copy(uidx_hbm.at[pl.ds(base, BPW)], uidx_v)
    pltpu.sync_copy(gidx_hbm.at[pl.ds(base, BPW)], gidx_v)
    pltpu.sync_copy(yidx_hbm.at[pl.ds(base, BPW)], yidx_v)
    pltpu.sync_copy(proj_hbm, proj_v)

    for j in range(NCHUNK):
        for k in range(CROWS // L):
            hi_v[j, pl.ds(k * L, L)] = lax.shift_right_logical(
                mflat_v[pl.ds(j * CROWS + k * L, L)], 4)

    descs = [
        pltpu.async_copy(p_hbm.at[hi_v.at[j]],
                         prows_v.at[pl.ds(j * CROWS, CROWS)], sem)
        for j in range(NCHUNK)
    ]
    for d in descs:
        d.wait()

    riota = lax.iota(jnp.int32, L)

    def group(g, carry):
        row0 = g * L
        rvec = riota + row0
        lovec = mflat_v[pl.ds(row0, L)] & 15
        acc = plsc.load_gather(prows_v, [rvec, lovec])
        uvec = uidx_v[pl.ds(row0, L)] + UOFF
        gvec = gidx_v[pl.ds(row0, L)] + GOFF
        yvec = yidx_v[pl.ds(row0, L)] + YOFF
        acc = acc + plsc.load_gather(proj_v, [uvec])
        acc = acc + plsc.load_gather(proj_v, [gvec])
        acc = acc + plsc.load_gather(proj_v, [yvec])
        out_v[pl.ds(row0, L)] = acc
        return carry

    lax.fori_loop(0, NGROUPS, group, 0)
    pltpu.sync_copy(out_v, out_hbm.at[pl.ds(base, BPW)])


def kernel(user, movie, genre, year, user_table, movie_table, genre_table,
           year_table, fc_w, fc_b):
    user = user.astype(jnp.int32)
    movie = movie.astype(jnp.int32)
    genre = genre.astype(jnp.int32)
    year = year.astype(jnp.int32)

    p, proj2 = pl.pallas_call(
        _tc_mv_body,
        grid=(MV_GRID,),
        in_specs=[
            pl.BlockSpec((1, 256), lambda i: (0, 0)),
            pl.BlockSpec((EMB, MV_BLK), lambda i: (0, i)),
            pl.BlockSpec((EMB, 5), lambda i: (0, 0)),
            pl.BlockSpec((EMB, 31), lambda i: (0, 0)),
            pl.BlockSpec((EMB, 101), lambda i: (0, 0)),
            pl.BlockSpec((1, 1), lambda i: (0, 0)),
        ],
        out_specs=[
            pl.BlockSpec((1, MV_BLK), lambda i: (0, i)),
            pl.BlockSpec((1, PROJ_N), lambda i: (0, 0)),
        ],
        out_shape=[
            jax.ShapeDtypeStruct((1, MV_GRID * MV_BLK), jnp.float32),
            jax.ShapeDtypeStruct((1, PROJ_N), jnp.float32),
        ],
    )(fc_w.reshape(1, 256), movie_table.T, user_table.T, genre_table.T,
      year_table.T, fc_b.reshape(1, 1))
    p16 = p.reshape(MV_GRID * MV_BLK // 16, 16)
    proj = proj2.reshape(PROJ_N)

    sc = pl.kernel(
        _sc_body,
        mesh=plsc.VectorSubcoreMesh(**_SC_MESH),
        compiler_params=_SC_PARAMS,
        out_type=jax.ShapeDtypeStruct((B,), jnp.float32),
        scratch_types=[
            pltpu.VMEM((NCHUNK, CROWS), jnp.int32),
            pltpu.VMEM((BPW,), jnp.int32),
            pltpu.VMEM((BPW,), jnp.int32),
            pltpu.VMEM((BPW,), jnp.int32),
            pltpu.VMEM((BPW,), jnp.int32),
            pltpu.VMEM((PROJ_N,), jnp.float32),
            pltpu.VMEM((BPW, L), jnp.float32),
            pltpu.VMEM((BPW,), jnp.float32),
            pltpu.SemaphoreType.DMA,
        ],
    )
    out = sc(movie, user, genre, year, proj, p16)
    return out.reshape(B, 1)
